# Initial kernel scaffold; baseline (speedup 1.0000x reference)
import jax
import jax.numpy as jnp
from jax.experimental import pallas as pl


def kernel(x, attention_mask, W, b):
    raise NotImplementedError("write your pallas kernel here")


# speedup vs baseline: 1.6616x; 1.6616x over previous
import jax
import jax.numpy as jnp
from jax.experimental import pallas as pl

_HID = 768
_VOCAB = 100000
_B, _S = 64, 4
_TEMP = 0.7
_K = 50
_P = 0.9
_BV = 2048


def _proj_kernel(x_ref, w_ref, b_ref, o_ref):
    acc = jnp.dot(x_ref[...], w_ref[...], preferred_element_type=jnp.float32)
    o_ref[...] = (acc + b_ref[...]) * (1.0 / _TEMP)


def _projection(x2d, W, b2d):
    grid = (pl.cdiv(_VOCAB, _BV),)
    return pl.pallas_call(
        _proj_kernel,
        grid=grid,
        in_specs=[
            pl.BlockSpec((_B * _S, _HID), lambda i: (0, 0)),
            pl.BlockSpec((_HID, _BV), lambda i: (0, i)),
            pl.BlockSpec((1, _BV), lambda i: (0, i)),
        ],
        out_specs=pl.BlockSpec((_B * _S, _BV), lambda i: (0, i)),
        out_shape=jax.ShapeDtypeStruct((_B * _S, _VOCAB), jnp.float32),
    )(x2d, W, b2d)


def kernel(x, attention_mask, W, b):
    del attention_mask
    x2d = x.reshape(_B * _S, _HID)
    logits = _projection(x2d, W, b.reshape(1, _VOCAB))
    logits = logits.reshape(_B, _S, _VOCAB)

    top_vals, top_idx = jax.lax.top_k(logits, _K)

    probs = jax.nn.softmax(top_vals, axis=-1)
    cum = jnp.cumsum(probs, axis=-1)
    keep = cum <= _P
    filt = jnp.where(keep, top_vals, -jnp.inf)
    p_sorted = jax.nn.softmax(filt, axis=-1)

    j = jnp.arange(_K)
    eq = top_idx[..., None] == j
    in_top = jnp.any(eq, axis=-2)
    r_in = jnp.argmax(eq, axis=-2)
    cnt_less = jnp.sum(top_idx[..., None] < j, axis=-2)
    ranks = jnp.where(in_top, r_in, _K + j - cnt_less)

    onehot = jax.nn.one_hot(ranks, 2 * _K, dtype=jnp.float32)
    small = jnp.einsum('bsk,bskc->bsc', p_sorted, onehot)
    out = jnp.pad(small, ((0, 0), (0, 0), (0, _VOCAB - 2 * _K)))
    return out
